# baseline (speedup 1.0000x reference)
import functools

import jax
import jax.numpy as jnp
import numpy as np
from jax import lax
from jax.experimental import pallas as pl
from jax.experimental.pallas import tpu as pltpu
from jax.experimental.pallas import tpu_sc as plsc

_LEVEL_W = (128, 64, 32, 16)
_NL = 4
_NC = 2
_NS = 16
_NW = _NC * _NS


def _prep_body(pts_ref, cam_ref, img_ref, *out_refs, V):
    idx_refs = out_refs[:_NL]
    wgt_refs = out_refs[_NL:2 * _NL]
    cnt_ref = out_refs[2 * _NL]
    b = pl.program_id(0)
    v = pl.program_id(1)
    x = pts_ref[0, 0]
    y = pts_ref[0, 1]
    z3 = pts_ref[0, 2]

    def c(k):
        return cam_ref[b, v, k]

    Xc = x * c(0) + y * c(1) + z3 * c(2) + c(9)
    Yc = x * c(3) + y * c(4) + z3 * c(5) + c(10)
    Zc = x * c(6) + y * c(7) + z3 * c(8) + c(11)
    fx, fy, cx, cy = c(12), c(13), c(14), c(15)
    zc = jnp.maximum(Zc, 0.1)
    Hf = img_ref[0]
    Wf = img_ref[1]
    u = Xc * fx / zc + cx
    vv = Yc * fy / zc + cy
    u_n = 2.0 * u / (Wf - 1.0) - 1.0
    v_n = 2.0 * vv / (Hf - 1.0) - 1.0
    in_view = (u_n > -1.0) & (u_n < 1.0) & (v_n > -1.0) & (v_n < 1.0) & (Zc > 0.0)
    ivf = in_view.astype(jnp.float32)

    @pl.when(v == 0)
    def _():
        cnt_ref[0] = ivf

    @pl.when(v != 0)
    def _():
        cnt_ref[0] += ivf

    for l, Wl in enumerate(_LEVEL_W):
        base = (b * V + v) * (Wl * Wl)
        ix = ((u_n + 1.0) * Wl - 1.0) / 2.0
        iy = ((v_n + 1.0) * Wl - 1.0) / 2.0
        ix0 = jnp.floor(ix)
        iy0 = jnp.floor(iy)
        wx1 = ix - ix0
        wx0 = 1.0 - wx1
        wy1 = iy - iy0
        wy0 = 1.0 - wy1
        r = jnp.clip(ix0, 0.0, Wl - 2.0)
        vl = ((ix0 >= 0.0) & (ix0 <= Wl - 1.0)).astype(jnp.float32)
        vr = ((ix0 >= -1.0) & (ix0 <= Wl - 2.0)).astype(jnp.float32)
        o_l = jnp.clip(ix0, 0.0, Wl - 1.0) - r
        o_r = jnp.clip(ix0 + 1.0, 0.0, Wl - 1.0) - r
        ri = r.astype(jnp.int32)
        for yi, (yq, wy) in enumerate(((iy0, wy0), (iy0 + 1.0, wy1))):
            vy = ((yq >= 0.0) & (yq <= Wl - 1.0)).astype(jnp.float32)
            yc = jnp.clip(yq, 0.0, Wl - 1.0).astype(jnp.int32)
            wL = 0.25 * wx0 * wy * vl * vy * ivf
            wR = 0.25 * wx1 * wy * vr * vy * ivf
            idx_refs[l][yi, 0] = base + yc * Wl + ri
            wgt_refs[l][2 * yi, 0] = (jnp.where(o_l < 0.5, wL, 0.0)
                                      + jnp.where(o_r < 0.5, wR, 0.0))
            wgt_refs[l][2 * yi + 1, 0] = (jnp.where(o_l >= 0.5, wL, 0.0)
                                          + jnp.where(o_r >= 0.5, wR, 0.0))


def _prep(pts, cam, img, B, V, nqr):
    idx_spec = pl.BlockSpec((2, 1, nqr, 128), lambda b, v: (v, b, 0, 0))
    wgt_spec = pl.BlockSpec((4, 1, nqr, 128), lambda b, v: (v, b, 0, 0))
    idx_shape = jax.ShapeDtypeStruct((V * 2, B, nqr, 128), jnp.int32)
    wgt_shape = jax.ShapeDtypeStruct((V * 4, B, nqr, 128), jnp.float32)
    return pl.pallas_call(
        functools.partial(_prep_body, V=V),
        grid=(B, V),
        in_specs=[
            pl.BlockSpec((1, 3, nqr, 128), lambda b, v: (b, 0, 0, 0)),
            pl.BlockSpec(memory_space=pltpu.SMEM),
            pl.BlockSpec(memory_space=pltpu.SMEM),
        ],
        out_specs=tuple([idx_spec] * _NL + [wgt_spec] * _NL
                        + [pl.BlockSpec((1, nqr, 128), lambda b, v: (b, 0, 0))]),
        out_shape=tuple([idx_shape] * _NL + [wgt_shape] * _NL
                        + [jax.ShapeDtypeStruct((B, nqr, 128), jnp.float32)]),
    )(pts, cam, img)


def _bcast_lane(vec, jj):
    idx = jnp.full((16, 1), jj, jnp.int32)
    return lax.gather(
        vec, idx,
        dimension_numbers=lax.GatherDimensionNumbers(
            offset_dims=(), collapsed_slice_dims=(0,), start_index_map=(0,)),
        slice_sizes=(1,),
        mode=lax.GatherScatterMode.PROMISE_IN_BOUNDS)


def _sc_gather_accumulate(tables, idxs, wgts, C):
    nq_tot, JL = idxs[0].shape
    J = _NL * JL
    qpw = nq_tot // _NW
    QB = 64
    OB = 8
    nchunk = C // 16
    mesh = plsc.VectorSubcoreMesh(core_axis_name="c", subcore_axis_name="s")

    @functools.partial(
        pl.kernel,
        out_type=jax.ShapeDtypeStruct((nq_tot, C), jnp.float32),
        mesh=mesh,
        scratch_types=[
            [pltpu.VMEM((QB, JL), jnp.int32) for _ in range(_NL)],
            [pltpu.VMEM((QB, 2 * JL), jnp.float32) for _ in range(_NL)],
            pltpu.VMEM((J, C), jnp.int32),
            pltpu.VMEM((J, C), jnp.int32),
            pltpu.VMEM((OB, C), jnp.float32),
            pltpu.SemaphoreType.DMA,
            pltpu.SemaphoreType.DMA,
        ],
    )
    def run(t0, t1, t2, t3, i0, i1, i2, i3, w0, w1, w2, w3, out_hbm,
            idx_v, w_v, buf0, buf1, ost, sem0, sem1):
        tabs = (t0, t1, t2, t3)
        ihbm = (i0, i1, i2, i3)
        whbm = (w0, w1, w2, w3)
        wid = lax.axis_index("s") * _NC + lax.axis_index("c")
        base = wid * qpw

        def start(q, buf, sem):
            for l in range(_NL):
                pltpu.make_async_copy(
                    tabs[l].at[idx_v[l].at[q]],
                    buf.at[pl.ds(l * JL, JL)],
                    sem).start()

        def wait(buf, sem):
            pltpu.make_async_copy(tabs[0].at[pl.ds(0, J)], buf, sem).wait()

        def accum(q, buf):
            accs = tuple(jnp.zeros((16,), jnp.float32) for _ in range(nchunk))
            for l in range(_NL):
                for tj in range(JL // 8):
                    wrow = w_v[l][q, pl.ds(16 * tj, 16)]

                    def jjbody(jj, accs, l=l, tj=tj, wrow=wrow):
                        wl = _bcast_lane(wrow, 2 * jj)
                        wr = _bcast_lane(wrow, 2 * jj + 1)
                        j = l * JL + 8 * tj + jj
                        new = []
                        for t in range(C // 32):
                            cwl = buf[j, pl.ds(16 * t, 16)]
                            cwr = buf[j, pl.ds(C // 2 + 16 * t, 16)]
                            lo_l = lax.bitcast_convert_type(
                                cwl << 16, jnp.float32)
                            hi_l = lax.bitcast_convert_type(
                                cwl & jnp.int32(-65536), jnp.float32)
                            lo_r = lax.bitcast_convert_type(
                                cwr << 16, jnp.float32)
                            hi_r = lax.bitcast_convert_type(
                                cwr & jnp.int32(-65536), jnp.float32)
                            new.append(accs[2 * t] + wl * lo_l + wr * lo_r)
                            new.append(accs[2 * t + 1] + wl * hi_l + wr * hi_r)
                        return tuple(new)

                    accs = lax.fori_loop(0, 8, jjbody, accs, unroll=4)
            qm = lax.rem(q, OB)
            for t in range(nchunk):
                ost[qm, pl.ds(16 * t, 16)] = accs[t]

        def blkbody(bi, carry):
            qb = bi * QB
            for l in range(_NL):
                pltpu.sync_copy(ihbm[l].at[pl.ds(base + qb, QB)], idx_v[l])
                pltpu.sync_copy(whbm[l].at[pl.ds(base + qb, QB)], w_v[l])
            start(0, buf0, sem0)
            start(1, buf1, sem1)

            def qbody(i, carry):
                q0 = 2 * i
                wait(buf0, sem0)
                accum(q0, buf0)

                @pl.when(q0 + 2 < QB)
                def _():
                    start(q0 + 2, buf0, sem0)

                wait(buf1, sem1)
                accum(q0 + 1, buf1)

                @pl.when(q0 + 3 < QB)
                def _():
                    start(q0 + 3, buf1, sem1)

                @pl.when(lax.rem(q0, OB) == OB - 2)
                def _():
                    pltpu.sync_copy(
                        ost,
                        out_hbm.at[pl.ds(base + qb + (q0 // OB) * OB, OB)])

                return carry

            return lax.fori_loop(0, QB // 2, qbody, carry)

        lax.fori_loop(0, qpw // QB, blkbody, 0)

    return run(*tables, *idxs, *wgts)


def _finish_body(q_ref, acc_ref, cnt_ref, wt_ref, b_ref, g_ref, be_ref, o_ref):
    f = acc_ref[...] / jnp.maximum(cnt_ref[...], 1.0)
    o = q_ref[...] + jnp.dot(f, wt_ref[...],
                             preferred_element_type=jnp.float32) + b_ref[...]
    mu = jnp.mean(o, axis=-1, keepdims=True)
    d = o - mu
    var = jnp.mean(d * d, axis=-1, keepdims=True)
    o_ref[...] = d * lax.rsqrt(var + 1e-5) * g_ref[...] + be_ref[...]


def _finish(q2, acc, cnt2, w_t, b2, g2, be2, blk=512):
    n, C = q2.shape
    return pl.pallas_call(
        _finish_body,
        grid=(n // blk,),
        in_specs=[
            pl.BlockSpec((blk, C), lambda i: (i, 0)),
            pl.BlockSpec((blk, C), lambda i: (i, 0)),
            pl.BlockSpec((blk, 1), lambda i: (i, 0)),
            pl.BlockSpec((C, C), lambda i: (0, 0)),
            pl.BlockSpec((1, C), lambda i: (0, 0)),
            pl.BlockSpec((1, C), lambda i: (0, 0)),
            pl.BlockSpec((1, C), lambda i: (0, 0)),
        ],
        out_specs=pl.BlockSpec((blk, C), lambda i: (i, 0)),
        out_shape=jax.ShapeDtypeStruct((n, C), jnp.float32),
    )(q2, acc, cnt2, w_t, b2, g2, be2)


def kernel(query, reference_points_3d, feats_l0, feats_l1, feats_l2, feats_l3,
           camera_R, camera_T, camera_K, W_out, b_out, ln_gamma, ln_beta,
           img_h, img_w):
    B, Nq, C = query.shape
    V = camera_R.shape[1]
    nqr = Nq // 128

    tables = []
    for f in (feats_l0, feats_l1, feats_l2, feats_l3):
        H, W = f.shape[3], f.shape[4]
        t = f.transpose(0, 1, 3, 4, 2).reshape(B * V * H * W, C)
        tb = t.astype(jnp.bfloat16)
        tb1 = jnp.concatenate(
            [tb[1:], jnp.zeros((1, C), jnp.bfloat16)], axis=0)
        pair = jnp.concatenate([tb, tb1], axis=1).reshape(-1, C, 2)
        tables.append(lax.bitcast_convert_type(pair, jnp.int32))

    pts = reference_points_3d.transpose(0, 2, 1).reshape(B, 3, nqr, 128)
    Rr = camera_R.reshape(B, V, 9)
    Kf = jnp.stack([camera_K[..., 0, 0], camera_K[..., 1, 1],
                    camera_K[..., 0, 2], camera_K[..., 1, 2]], axis=-1)
    cam = jnp.concatenate([Rr, camera_T, Kf], axis=-1)
    img = jnp.stack([jnp.float32(img_h), jnp.float32(img_w)])

    outs = _prep(pts, cam, img, B, V, nqr)
    idxs = [o.reshape(V * 2, B * Nq).T for o in outs[:_NL]]
    wgts = [o.reshape(V * 4, B * Nq).T for o in outs[_NL:2 * _NL]]
    cnt = outs[2 * _NL]

    acc = _sc_gather_accumulate(tables, idxs, wgts, C)

    p = np.empty((C,), np.int32)
    for t in range(C // 32):
        p[32 * t:32 * t + 16] = 32 * t + np.arange(0, 32, 2)
        p[32 * t + 16:32 * t + 32] = 32 * t + np.arange(1, 32, 2)
    W_t = W_out.T[jnp.asarray(p)]

    out = _finish(query.reshape(B * Nq, C), acc, cnt.reshape(B * Nq, 1),
                  W_t, b_out.reshape(1, C), ln_gamma.reshape(1, C),
                  ln_beta.reshape(1, C))
    return out.reshape(B, Nq, C)


# speedup vs baseline: 4.5752x; 4.5752x over previous
import functools

import jax
import jax.numpy as jnp
import numpy as np
from jax import lax
from jax.experimental import pallas as pl
from jax.experimental.pallas import tpu as pltpu
from jax.experimental.pallas import tpu_sc as plsc

_LEVEL_W = (128, 64, 32, 16)
_NL = 4
_NC = 2
_NS = 16
_NW = _NC * _NS


def _prep_body(pts_ref, cam_ref, img_ref, *out_refs, V):
    idx_refs = out_refs[:_NL]
    wgt_refs = out_refs[_NL:2 * _NL]
    cnt_ref = out_refs[2 * _NL]
    b = pl.program_id(0)
    v = pl.program_id(1)
    x = pts_ref[0, 0]
    y = pts_ref[0, 1]
    z3 = pts_ref[0, 2]

    def c(k):
        return cam_ref[b, v, k]

    Xc = x * c(0) + y * c(1) + z3 * c(2) + c(9)
    Yc = x * c(3) + y * c(4) + z3 * c(5) + c(10)
    Zc = x * c(6) + y * c(7) + z3 * c(8) + c(11)
    fx, fy, cx, cy = c(12), c(13), c(14), c(15)
    zc = jnp.maximum(Zc, 0.1)
    Hf = img_ref[0]
    Wf = img_ref[1]
    u = Xc * fx / zc + cx
    vv = Yc * fy / zc + cy
    u_n = 2.0 * u / (Wf - 1.0) - 1.0
    v_n = 2.0 * vv / (Hf - 1.0) - 1.0
    in_view = (u_n > -1.0) & (u_n < 1.0) & (v_n > -1.0) & (v_n < 1.0) & (Zc > 0.0)
    ivf = in_view.astype(jnp.float32)

    @pl.when(v == 0)
    def _():
        cnt_ref[0] = ivf

    @pl.when(v != 0)
    def _():
        cnt_ref[0] += ivf

    for l, Wl in enumerate(_LEVEL_W):
        base = (b * V + v) * (Wl * Wl)
        ix = ((u_n + 1.0) * Wl - 1.0) / 2.0
        iy = ((v_n + 1.0) * Wl - 1.0) / 2.0
        ix0 = jnp.floor(ix)
        iy0 = jnp.floor(iy)
        wx1 = ix - ix0
        wx0 = 1.0 - wx1
        wy1 = iy - iy0
        wy0 = 1.0 - wy1
        corners = ((ix0, iy0, wx0 * wy0), (ix0 + 1.0, iy0, wx1 * wy0),
                   (ix0, iy0 + 1.0, wx0 * wy1), (ix0 + 1.0, iy0 + 1.0, wx1 * wy1))
        for ci, (xq, yq, wq) in enumerate(corners):
            valid = ((xq >= 0.0) & (xq <= Wl - 1.0)
                     & (yq >= 0.0) & (yq <= Wl - 1.0))
            ixc = jnp.clip(xq, 0.0, Wl - 1.0).astype(jnp.int32)
            iyc = jnp.clip(yq, 0.0, Wl - 1.0).astype(jnp.int32)
            row = base + iyc * Wl + ixc
            w = 0.25 * wq * valid.astype(jnp.float32) * ivf
            idx_refs[l][ci, 0] = row
            wgt_refs[l][ci, 0] = w


def _prep(pts, cam, img, B, V, nqr):
    idx_spec = pl.BlockSpec((4, 1, nqr, 128), lambda b, v: (v, b, 0, 0))
    wgt_spec = pl.BlockSpec((4, 1, nqr, 128), lambda b, v: (v, b, 0, 0))
    idx_shape = jax.ShapeDtypeStruct((V * 4, B, nqr, 128), jnp.int32)
    wgt_shape = jax.ShapeDtypeStruct((V * 4, B, nqr, 128), jnp.float32)
    return pl.pallas_call(
        functools.partial(_prep_body, V=V),
        grid=(B, V),
        in_specs=[
            pl.BlockSpec((1, 3, nqr, 128), lambda b, v: (b, 0, 0, 0)),
            pl.BlockSpec(memory_space=pltpu.SMEM),
            pl.BlockSpec(memory_space=pltpu.SMEM),
        ],
        out_specs=tuple([idx_spec] * _NL + [wgt_spec] * _NL
                        + [pl.BlockSpec((1, nqr, 128), lambda b, v: (b, 0, 0))]),
        out_shape=tuple([idx_shape] * _NL + [wgt_shape] * _NL
                        + [jax.ShapeDtypeStruct((B, nqr, 128), jnp.float32)]),
    )(pts, cam, img)


def _bcast_lane(vec, jj):
    idx = jnp.full((16, 1), jj, jnp.int32)
    return lax.gather(
        vec, idx,
        dimension_numbers=lax.GatherDimensionNumbers(
            offset_dims=(), collapsed_slice_dims=(0,), start_index_map=(0,)),
        slice_sizes=(1,),
        mode=lax.GatherScatterMode.PROMISE_IN_BOUNDS)


def _sc_gather_accumulate(tables, idxs, wgts, C):
    nq_tot, JL = idxs[0].shape
    J = _NL * JL
    RL = JL // 2
    JR = _NL * RL
    qpw = nq_tot // _NW
    QB = 64
    OB = 8
    nchunk = C // 16
    mesh = plsc.VectorSubcoreMesh(core_axis_name="c", subcore_axis_name="s")

    @functools.partial(
        pl.kernel,
        out_type=jax.ShapeDtypeStruct((nq_tot, C), jnp.float32),
        mesh=mesh,
        scratch_types=[
            [pltpu.VMEM((QB, JL), jnp.int32) for _ in range(_NL)],
            [pltpu.VMEM((QB, JL), jnp.float32) for _ in range(_NL)],
            pltpu.VMEM((J, C), jnp.float32),
            pltpu.VMEM((J, C), jnp.float32),
            pltpu.VMEM((OB, C), jnp.float32),
            pltpu.SemaphoreType.DMA,
            pltpu.SemaphoreType.DMA,
        ],
    )
    def run(t0, t1, t2, t3, i0, i1, i2, i3, w0, w1, w2, w3, out_hbm,
            idx_v, w_v, buf0, buf1, ost, sem0, sem1):
        tabs = (t0, t1, t2, t3)
        ihbm = (i0, i1, i2, i3)
        whbm = (w0, w1, w2, w3)
        wid = lax.axis_index("s") * _NC + lax.axis_index("c")
        base = wid * qpw

        def start(q, buf, sem):
            for l in range(_NL):
                pltpu.make_async_copy(
                    tabs[l].at[idx_v[l].at[q]],
                    buf.at[pl.ds(l * JL, JL)],
                    sem).start()

        def wait(buf, sem):
            pltpu.make_async_copy(tabs[0].at[pl.ds(0, J)], buf, sem).wait()

        def accum(q, buf):
            accs = tuple(jnp.zeros((16,), jnp.float32) for _ in range(nchunk))
            for l in range(_NL):
                for tj in range(JL // 16):
                    wrow = w_v[l][q, pl.ds(16 * tj, 16)]

                    def jjbody(jj, accs, l=l, tj=tj, wrow=wrow):
                        wb = _bcast_lane(wrow, jj)
                        j = l * JL + 16 * tj + jj
                        return tuple(
                            accs[t] + wb * buf[j, pl.ds(16 * t, 16)]
                            for t in range(nchunk))

                    accs = lax.fori_loop(0, 16, jjbody, accs, unroll=4)
            qm = lax.rem(q, OB)
            for t in range(nchunk):
                ost[qm, pl.ds(16 * t, 16)] = accs[t]

        def blkbody(bi, carry):
            qb = bi * QB
            for l in range(_NL):
                pltpu.sync_copy(ihbm[l].at[pl.ds(base + qb, QB)], idx_v[l])
                pltpu.sync_copy(whbm[l].at[pl.ds(base + qb, QB)], w_v[l])
            start(0, buf0, sem0)
            start(1, buf1, sem1)

            def qbody(i, carry):
                q0 = 2 * i
                wait(buf0, sem0)
                accum(q0, buf0)

                @pl.when(q0 + 2 < QB)
                def _():
                    start(q0 + 2, buf0, sem0)

                wait(buf1, sem1)
                accum(q0 + 1, buf1)

                @pl.when(q0 + 3 < QB)
                def _():
                    start(q0 + 3, buf1, sem1)

                @pl.when(lax.rem(q0, OB) == OB - 2)
                def _():
                    pltpu.sync_copy(
                        ost,
                        out_hbm.at[pl.ds(base + qb + (q0 // OB) * OB, OB)])

                return carry

            return lax.fori_loop(0, QB // 2, qbody, carry)

        lax.fori_loop(0, qpw // QB, blkbody, 0)

    return run(*tables, *idxs, *wgts)


def _finish_body(q_ref, acc_ref, cnt_ref, wt_ref, b_ref, g_ref, be_ref, o_ref):
    f = acc_ref[...] / jnp.maximum(cnt_ref[...], 1.0)
    o = q_ref[...] + jnp.dot(f, wt_ref[...],
                             preferred_element_type=jnp.float32) + b_ref[...]
    mu = jnp.mean(o, axis=-1, keepdims=True)
    d = o - mu
    var = jnp.mean(d * d, axis=-1, keepdims=True)
    o_ref[...] = d * lax.rsqrt(var + 1e-5) * g_ref[...] + be_ref[...]


def _finish(q2, acc, cnt2, w_t, b2, g2, be2, blk=512):
    n, C = q2.shape
    return pl.pallas_call(
        _finish_body,
        grid=(n // blk,),
        in_specs=[
            pl.BlockSpec((blk, C), lambda i: (i, 0)),
            pl.BlockSpec((blk, C), lambda i: (i, 0)),
            pl.BlockSpec((blk, 1), lambda i: (i, 0)),
            pl.BlockSpec((C, C), lambda i: (0, 0)),
            pl.BlockSpec((1, C), lambda i: (0, 0)),
            pl.BlockSpec((1, C), lambda i: (0, 0)),
            pl.BlockSpec((1, C), lambda i: (0, 0)),
        ],
        out_specs=pl.BlockSpec((blk, C), lambda i: (i, 0)),
        out_shape=jax.ShapeDtypeStruct((n, C), jnp.float32),
    )(q2, acc, cnt2, w_t, b2, g2, be2)


def kernel(query, reference_points_3d, feats_l0, feats_l1, feats_l2, feats_l3,
           camera_R, camera_T, camera_K, W_out, b_out, ln_gamma, ln_beta,
           img_h, img_w):
    B, Nq, C = query.shape
    V = camera_R.shape[1]
    nqr = Nq // 128

    tables = []
    for f in (feats_l0, feats_l1, feats_l2, feats_l3):
        H, W = f.shape[3], f.shape[4]
        t = f.transpose(0, 1, 3, 4, 2).reshape(B * V * H * W, C)
        tables.append(t)

    pts = reference_points_3d.transpose(0, 2, 1).reshape(B, 3, nqr, 128)
    Rr = camera_R.reshape(B, V, 9)
    Kf = jnp.stack([camera_K[..., 0, 0], camera_K[..., 1, 1],
                    camera_K[..., 0, 2], camera_K[..., 1, 2]], axis=-1)
    cam = jnp.concatenate([Rr, camera_T, Kf], axis=-1)
    img = jnp.stack([jnp.float32(img_h), jnp.float32(img_w)])

    outs = _prep(pts, cam, img, B, V, nqr)
    idxs = [o.reshape(V * 4, B * Nq).T for o in outs[:_NL]]
    wgts = [o.reshape(V * 4, B * Nq).T for o in outs[_NL:2 * _NL]]
    cnt = outs[2 * _NL]

    acc = _sc_gather_accumulate(tables, idxs, wgts, C)

    out = _finish(query.reshape(B * Nq, C), acc, cnt.reshape(B * Nq, 1),
                  W_out.T, b_out.reshape(1, C), ln_gamma.reshape(1, C),
                  ln_beta.reshape(1, C))
    return out.reshape(B, Nq, C)
